# baseline (speedup 1.0000x reference)
import functools

import jax
import jax.numpy as jnp
from jax import lax
from jax.experimental import pallas as pl
from jax.experimental.pallas import tpu as pltpu
from jax.experimental.pallas import tpu_sc as plsc

_L = 200
_B = 4096
_D = 64
_TB = 128


def _build_kernel(V):
    info = plsc.get_sparse_core_info()
    NC, NS = info.num_cores, info.num_subcores
    NW = NC * NS
    NTC = _B // _TB
    UNITS = _L * NTC
    per_w = UNITS // NW
    assert UNITS % NW == 0 and per_w % 2 == 0

    mesh = plsc.VectorSubcoreMesh(core_axis_name="c", subcore_axis_name="s")

    @functools.partial(
        pl.kernel,
        mesh=mesh,
        out_type=jax.ShapeDtypeStruct((_L, _D // 8, NTC, 8, _TB), jnp.float32),
        compiler_params=pltpu.CompilerParams(
            use_tc_tiling_on_sc=False, needs_layout_passes=False),
        scratch_types=[
            pltpu.VMEM((_TB,), jnp.int32),
            pltpu.VMEM((_TB,), jnp.int32),
            pltpu.VMEM((_TB, _D), jnp.float32),
            pltpu.VMEM((_TB, _D), jnp.float32),
            pltpu.VMEM((_D // 8, 8, _TB), jnp.float32),
            pltpu.VMEM((_D // 8, 8, _TB), jnp.float32),
            pltpu.VMEM((_L * _D,), jnp.float32),
            pltpu.SemaphoreType.DMA,
            pltpu.SemaphoreType.DMA,
            pltpu.SemaphoreType.DMA,
            pltpu.SemaphoreType.DMA,
            pltpu.SemaphoreType.DMA,
            pltpu.SemaphoreType.DMA,
        ],
    )
    def k(idx_hbm, tab_hbm, pos_hbm, out_hbm,
          idx0, idx1, rows0, rows1, tr0, tr1, pos_v,
          isem0, isem1, gsem0, gsem1, ssem0, ssem1):
        wid = lax.axis_index("s") * NC + lax.axis_index("c")
        u_base = wid * per_w

        idx_bufs = (idx0, idx1)
        rows_bufs = (rows0, rows1)
        tr_bufs = (tr0, tr1)
        isems = (isem0, isem1)
        gsems = (gsem0, gsem1)
        ssems = (ssem0, ssem1)

        pltpu.sync_copy(pos_hbm, pos_v)

        iota16 = lax.iota(jnp.int32, 16)
        row_gs = [iota16 + 16 * g for g in range(8)]

        def unit_lc(u):
            U = u_base + u
            return U // NTC, U % NTC

        def start_idx(u, b):
            l, tc = unit_lc(u)
            return pltpu.async_copy(
                idx_hbm.at[pl.ds(l * _B + tc * _TB, _TB)], idx_bufs[b],
                isems[b])

        def wait_idx(u, b):
            l, tc = unit_lc(u)
            pltpu.make_async_copy(
                idx_hbm.at[pl.ds(l * _B + tc * _TB, _TB)], idx_bufs[b],
                isems[b]).wait()

        def start_gather(b):
            return pltpu.async_copy(
                tab_hbm.at[idx_bufs[b]], rows_bufs[b], gsems[b])

        def wait_gather(b):
            pltpu.make_async_copy(
                tab_hbm.at[idx_bufs[b]], rows_bufs[b], gsems[b]).wait()

        def transpose_add(u, b):
            l, _ = unit_lc(u)
            rows = rows_bufs[b]
            trn = tr_bufs[b]

            def body(d, _):
                pidx = jnp.full((16,), l * _D + d, jnp.int32)
                pv = plsc.load_gather(pos_v, [pidx])
                col = jnp.full((16,), d, jnp.int32)
                t = d // 8
                s = d % 8
                for g in range(8):
                    v = plsc.load_gather(rows, [row_gs[g], col])
                    trn[t, s, pl.ds(16 * g, 16)] = v + pv
                return 0

            lax.fori_loop(0, _D, body, 0)

        def start_scatter(u, b):
            l, tc = unit_lc(u)
            return pltpu.async_copy(
                tr_bufs[b], out_hbm.at[l, :, tc], ssems[b])

        def wait_scatter(u, b):
            l, tc = unit_lc(u)
            pltpu.make_async_copy(
                tr_bufs[b], out_hbm.at[l, :, tc], ssems[b]).wait()

        wid0 = start_idx(0, 0)
        wid0.wait()
        start_gather(0)
        start_idx(1, 1)

        def pair(p, _):
            for b in (0, 1):
                u = 2 * p + b

                @pl.when(u + 1 < per_w)
                def _():
                    wait_idx(u + 1, 1 - b)
                    start_gather(1 - b)

                wait_gather(b)

                @pl.when(u + 2 < per_w)
                def _():
                    start_idx(u + 2, b)

                @pl.when(u >= 2)
                def _():
                    wait_scatter(u - 2, b)

                transpose_add(u, b)
                start_scatter(u, b)
            return 0

        lax.fori_loop(0, per_w // 2, pair, 0)
        wait_scatter(per_w - 2, 0)
        wait_scatter(per_w - 1, 1)

    return k


def kernel(inputs, token_table, pos_table):
    B, L = inputs.shape
    V, D = token_table.shape
    idx_t = jnp.transpose(inputs).reshape(L * B).astype(jnp.int32)
    pos_flat = pos_table.reshape(L * D)
    k = _build_kernel(V)
    out5 = k(idx_t, token_table, pos_flat)
    return out5.transpose(2, 4, 0, 1, 3).reshape(B, L, D)
